# baseline (speedup 1.0000x reference)
import functools

import jax
import jax.numpy as jnp
from jax import lax
from jax.experimental import pallas as pl
from jax.experimental.pallas import tpu as pltpu
from jax.experimental.pallas import tpu_sc as plsc

N = 10000
E = 320000
D = 128
NP = 10240
NC = 2
NS = 16
NW = NC * NS
EPW = E // NW
C = 128
NCHUNK = 80
EPW2 = NCHUNK * C
RPW = NP // NS
NROW = 2
NDI = 4


def _make_sc_segment_sum(with_cnt):
    mesh = plsc.VectorSubcoreMesh(core_axis_name="c", subcore_axis_name="s")

    out_type = [jax.ShapeDtypeStruct((NC, NP, D), jnp.float32)]
    scratch = [
        pltpu.VMEM_SHARED((NP, D), jnp.float32),
        pltpu.VMEM((EPW2,), jnp.int32),
        pltpu.VMEM((NDI, C), jnp.int32),
    ]
    scratch += [pltpu.VMEM((C, D), jnp.float32)] * NROW
    scratch += [pltpu.SemaphoreType.DMA] * (2 * NROW + NDI)
    if with_cnt:
        out_type.append(jax.ShapeDtypeStruct((NC, NP), jnp.float32))
        scratch += [
            pltpu.VMEM_SHARED((NP,), jnp.float32),
            pltpu.VMEM((RPW,), jnp.float32),
            pltpu.VMEM((C,), jnp.float32),
            pltpu.SemaphoreType.DMA,
        ]

    @functools.partial(pl.kernel, out_type=tuple(out_type), mesh=mesh,
                       scratch_types=scratch)
    def k(table_h, src_h, dst_h, agg_h, *rest):
        if with_cnt:
            cnt_h = rest[0]
            rest = rest[1:]
        aggs, sidx, didx = rest[0], rest[1], rest[2]
        rows = rest[3:3 + NROW]
        sems = rest[3 + NROW:3 + 3 * NROW + NDI]
        gsem, ssem, dsem = sems[:NROW], sems[NROW:2 * NROW], sems[2 * NROW:]
        if with_cnt:
            cnts, zvec, ones, csem = rest[3 + 3 * NROW + NDI:]
        cid = lax.axis_index("c")
        sid = lax.axis_index("s")
        wid = sid * NC + cid
        r0 = pl.multiple_of(sid * RPW, RPW)

        pltpu.sync_copy(src_h.at[wid], sidx)

        def _zrow(i, _):
            def _zcol(j, _):
                rows[0][i, pl.ds(j * 16, 16)] = jnp.zeros((16,), jnp.float32)
                return 0
            return lax.fori_loop(0, D // 16, _zcol, 0)
        lax.fori_loop(0, C, _zrow, 0)

        def _zs(kk, _):
            pltpu.sync_copy(rows[0], aggs.at[pl.ds(r0 + kk * C, C)])
            return 0
        lax.fori_loop(0, RPW // C, _zs, 0)

        if with_cnt:
            def _zv(i, _):
                zvec[pl.ds(i * 16, 16)] = jnp.zeros((16,), jnp.float32)
                return 0
            lax.fori_loop(0, RPW // 16, _zv, 0)

            def _ov(i, _):
                ones[pl.ds(i * 16, 16)] = jnp.ones((16,), jnp.float32)
                return 0
            lax.fori_loop(0, C // 16, _ov, 0)
            pltpu.sync_copy(zvec, cnts.at[pl.ds(r0, RPW)])

        plsc.subcore_barrier()

        def _fire_didx(kk, s):
            pltpu.async_copy(dst_h.at[wid, kk], didx.at[s], dsem[s])

        def _wait_didx(s):
            pltpu.make_async_copy(dst_h.at[wid, 0], didx.at[s],
                                  dsem[s]).wait()

        def _fire_gather(kk, b):
            pltpu.async_copy(table_h.at[sidx.at[pl.ds(kk * C, C)]],
                             rows[b], gsem[b])

        def _wait_gather(b):
            pltpu.make_async_copy(table_h.at[sidx.at[pl.ds(0, C)]], rows[b],
                                  gsem[b]).wait()

        def _fire_scatter(b, s):
            pltpu.async_copy(rows[b], aggs.at[didx.at[s]], ssem[b], add=True)
            if with_cnt:
                pltpu.async_copy(ones, cnts.at[didx.at[s]], csem, add=True)

        def _wait_scatter(b):
            pltpu.make_async_copy(rows[0], aggs.at[didx.at[0]],
                                  ssem[b]).wait()

        def _wait_cnt():
            pltpu.make_async_copy(ones, cnts.at[didx.at[0]], csem).wait()

        _fire_didx(0, 0)
        _fire_didx(1, 1)
        _fire_gather(0, 0)

        def _group(g, _):
            for j in range(NDI):
                kk = g * NDI + j
                r = j % NROW
                rn = (j + 1) % NROW

                if with_cnt:
                    @pl.when(kk >= 2)
                    def _():
                        _wait_cnt()

                @pl.when(kk >= 1)
                def _():
                    _wait_scatter(rn)

                @pl.when(kk + 2 < NCHUNK)
                def _():
                    _fire_didx(kk + 2, (j + 2) % NDI)

                @pl.when(kk + 1 < NCHUNK)
                def _():
                    _fire_gather(kk + 1, rn)
                _wait_gather(r)
                _wait_didx(j)
                _fire_scatter(r, j)
            return 0
        lax.fori_loop(0, NCHUNK // NDI, _group, 0)

        _wait_scatter((NCHUNK - 1) % NROW)
        if with_cnt:
            _wait_cnt()
            _wait_cnt()

        plsc.subcore_barrier()

        sl = pl.ds(r0, RPW)
        pltpu.sync_copy(aggs.at[sl], agg_h.at[cid].at[sl])
        if with_cnt:
            pltpu.sync_copy(cnts.at[sl], zvec)
            pltpu.sync_copy(zvec, cnt_h.at[cid].at[sl])

    return k


_sc_seg_cnt = _make_sc_segment_sum(True)
_sc_seg = _make_sc_segment_sum(False)


_BLK = 1024
_GRID = NP // _BLK


def _tc_layer1(aggp, cntp, xp, w1l, b1l, w1r):
    def body(agg_ref, cnt_ref, x_ref, wl_ref, bl_ref, wr_ref, o_ref):
        agg = agg_ref[0] + agg_ref[1]
        cnt = cnt_ref[0] + cnt_ref[1]
        inv = 1.0 / jnp.maximum(cnt, 1.0)
        mean = agg * inv[:, None]
        h = lax.dot_general(mean, wl_ref[...], (((1,), (1,)), ((), ())),
                            preferred_element_type=jnp.float32)
        h = h + bl_ref[...]
        h = h + lax.dot_general(x_ref[...], wr_ref[...], (((1,), (1,)), ((), ())),
                                preferred_element_type=jnp.float32)
        o_ref[...] = jnp.maximum(h, 0.0)

    return pl.pallas_call(
        body,
        grid=(_GRID,),
        in_specs=[
            pl.BlockSpec((NC, _BLK, D), lambda i: (0, i, 0)),
            pl.BlockSpec((NC, _BLK), lambda i: (0, i)),
            pl.BlockSpec((_BLK, D), lambda i: (i, 0)),
            pl.BlockSpec((D, D), lambda i: (0, 0)),
            pl.BlockSpec((1, D), lambda i: (0, 0)),
            pl.BlockSpec((D, D), lambda i: (0, 0)),
        ],
        out_specs=pl.BlockSpec((_BLK, D), lambda i: (i, 0)),
        out_shape=jax.ShapeDtypeStruct((NP, D), jnp.float32),
    )(aggp, cntp, xp, w1l, b1l, w1r)


def _tc_layer2(aggp, cntp, h1, w2l, b2l, w2r, wo1, bo1, wo2p, bo2p):
    def body(agg_ref, cnt_ref, h1_ref, wl_ref, bl_ref, wr_ref, wo1_ref,
             bo1_ref, wo2_ref, bo2_ref, o_ref):
        agg = agg_ref[0] + agg_ref[1]
        cnt = cnt_ref[0] + cnt_ref[1]
        inv = 1.0 / jnp.maximum(cnt, 1.0)
        mean = agg * inv[:, None]
        h = lax.dot_general(mean, wl_ref[...], (((1,), (1,)), ((), ())),
                            preferred_element_type=jnp.float32)
        h = h + bl_ref[...]
        h = h + lax.dot_general(h1_ref[...], wr_ref[...], (((1,), (1,)), ((), ())),
                                preferred_element_type=jnp.float32)
        h = jnp.maximum(h, 0.0)
        h = lax.dot_general(h, wo1_ref[...], (((1,), (1,)), ((), ())),
                            preferred_element_type=jnp.float32) + bo1_ref[...]
        h = jnp.maximum(h, 0.0)
        o_ref[...] = lax.dot_general(h, wo2_ref[...], (((1,), (1,)), ((), ())),
                                     preferred_element_type=jnp.float32) + bo2_ref[...]

    return pl.pallas_call(
        body,
        grid=(_GRID,),
        in_specs=[
            pl.BlockSpec((NC, _BLK, D), lambda i: (0, i, 0)),
            pl.BlockSpec((NC, _BLK), lambda i: (0, i)),
            pl.BlockSpec((_BLK, D), lambda i: (i, 0)),
            pl.BlockSpec((D, D), lambda i: (0, 0)),
            pl.BlockSpec((1, D), lambda i: (0, 0)),
            pl.BlockSpec((D, D), lambda i: (0, 0)),
            pl.BlockSpec((D, D), lambda i: (0, 0)),
            pl.BlockSpec((1, D), lambda i: (0, 0)),
            pl.BlockSpec((D, D), lambda i: (0, 0)),
            pl.BlockSpec((1, D), lambda i: (0, 0)),
        ],
        out_specs=pl.BlockSpec((_BLK, D), lambda i: (i, 0)),
        out_shape=jax.ShapeDtypeStruct((NP, D), jnp.float32),
    )(aggp, cntp, h1, w2l, b2l, w2r, wo1, bo1, wo2p, bo2p)


def kernel(x, edge_index, W1l, b1l, W1r, W2l, b2l, W2r, Wo1, bo1, Wo2, bo2):
    src2 = jnp.pad(edge_index[0].astype(jnp.int32).reshape(NW, EPW),
                   ((0, 0), (0, EPW2 - EPW)))
    dst3 = jnp.pad(edge_index[1].astype(jnp.int32).reshape(NW, EPW),
                   ((0, 0), (0, EPW2 - EPW)),
                   constant_values=NP - 1).reshape(NW, NCHUNK, C)
    xp = jnp.pad(x, ((0, NP - N), (0, 0)))

    wo2p = jnp.zeros((D, D), jnp.float32).at[:2].set(Wo2)
    bo2p = jnp.zeros((1, D), jnp.float32).at[0, :2].set(bo2)

    aggp1, cntp = _sc_seg_cnt(xp, src2, dst3)
    h1 = _tc_layer1(aggp1, cntp, xp, W1l, b1l.reshape(1, D), W1r)
    res2 = _sc_seg(h1, src2, dst3)
    aggp2 = res2[0] if isinstance(res2, (tuple, list)) else res2
    out = _tc_layer2(aggp2, cntp, h1, W2l, b2l.reshape(1, D), W2r,
                     Wo1, bo1.reshape(1, D), wo2p, bo2p)
    return out[:N, :2]


# speedup vs baseline: 2.7989x; 2.7989x over previous
import functools

import jax
import jax.numpy as jnp
from jax import lax
from jax.experimental import pallas as pl
from jax.experimental.pallas import tpu as pltpu
from jax.experimental.pallas import tpu_sc as plsc

N = 10000
E = 320000
D = 128
NP = 10240
NC = 2
NS = 16
NW = NC * NS
EPW = E // NW
C = 80
NCHUNK = EPW // C
EPW2 = EPW
RPW = NP // NS


def _make_sc_segment_sum(with_cnt):
    mesh = plsc.VectorSubcoreMesh(core_axis_name="c", subcore_axis_name="s")

    out_type = [jax.ShapeDtypeStruct((NC, NP, D), jnp.float32)]
    scratch = [
        pltpu.VMEM_SHARED((NP, D), jnp.float32),
        pltpu.VMEM((EPW,), jnp.int32),
        pltpu.VMEM((NCHUNK, C), jnp.int32),
    ]
    scratch += [pltpu.VMEM((C, D), jnp.float32)] * 2
    scratch += [pltpu.SemaphoreType.DMA] * 4
    if with_cnt:
        out_type.append(jax.ShapeDtypeStruct((NC, NP), jnp.float32))
        scratch += [
            pltpu.VMEM_SHARED((NP,), jnp.float32),
            pltpu.VMEM((RPW,), jnp.float32),
            pltpu.VMEM((C,), jnp.float32),
            pltpu.SemaphoreType.DMA,
        ]

    @functools.partial(pl.kernel, out_type=tuple(out_type), mesh=mesh,
                       scratch_types=scratch)
    def k(table_h, src_h, dst_h, agg_h, *rest):
        if with_cnt:
            cnt_h = rest[0]
            rest = rest[1:]
        aggs, sidx, didx = rest[0], rest[1], rest[2]
        rows = rest[3:5]
        gsem = rest[5:7]
        ssem = rest[7:9]
        if with_cnt:
            cnts, zvec, ones, csem = rest[9:]
        cid = lax.axis_index("c")
        sid = lax.axis_index("s")
        wid = sid * NC + cid
        r0 = pl.multiple_of(sid * RPW, RPW)

        pltpu.sync_copy(src_h.at[wid], sidx)

        def _zrow(i, _):
            def _zcol(j, _):
                rows[0][i, pl.ds(j * 16, 16)] = jnp.zeros((16,), jnp.float32)
                return 0
            return lax.fori_loop(0, D // 16, _zcol, 0)
        lax.fori_loop(0, C, _zrow, 0)

        def _zs(kk, _):
            pltpu.sync_copy(rows[0], aggs.at[pl.ds(r0 + kk * C, C)])
            return 0
        lax.fori_loop(0, RPW // C, _zs, 0)

        if with_cnt:
            def _zv(i, _):
                zvec[pl.ds(i * 16, 16)] = jnp.zeros((16,), jnp.float32)
                return 0
            lax.fori_loop(0, RPW // 16, _zv, 0)

            def _ov(i, _):
                ones[pl.ds(i * 16, 16)] = jnp.ones((16,), jnp.float32)
                return 0
            lax.fori_loop(0, C // 16, _ov, 0)
            pltpu.sync_copy(zvec, cnts.at[pl.ds(r0, RPW)])

        plsc.subcore_barrier()

        pltpu.sync_copy(dst_h.at[wid], didx)

        def _gather(kk, b):
            pltpu.async_copy(table_h.at[sidx.at[pl.ds(kk * C, C)]],
                             rows[b], gsem[b])

        def _wait_gather(b):
            pltpu.make_async_copy(table_h.at[sidx.at[pl.ds(0, C)]], rows[b],
                                  gsem[b]).wait()

        def _scatter(kk, b):
            pltpu.async_copy(rows[b], aggs.at[didx.at[kk]], ssem[b], add=True)
            if with_cnt:
                pltpu.async_copy(ones, cnts.at[didx.at[kk]], csem, add=True)

        def _wait_scatter(b):
            pltpu.make_async_copy(rows[0], aggs.at[didx.at[0]],
                                  ssem[b]).wait()

        _gather(0, 0)

        def _pair(g, _):
            k0 = g * 2
            @pl.when(g > 0)
            def _():
                _wait_scatter(1)
            _gather(k0 + 1, 1)
            _wait_gather(0)
            _scatter(k0, 0)
            _wait_scatter(0)
            _gather(k0 + 2, 0)
            _wait_gather(1)
            _scatter(k0 + 1, 1)
            return 0
        lax.fori_loop(0, NCHUNK // 2, _pair, 0)

        _wait_gather(0)
        _scatter(NCHUNK - 1, 0)
        _wait_scatter(0)
        _wait_scatter(1)
        if with_cnt:
            def _drain(kk, _):
                pltpu.make_async_copy(ones, cnts.at[didx.at[0]], csem).wait()
                return 0
            lax.fori_loop(0, NCHUNK, _drain, 0)

        plsc.subcore_barrier()

        sl = pl.ds(r0, RPW)
        pltpu.sync_copy(aggs.at[sl], agg_h.at[cid].at[sl])
        if with_cnt:
            pltpu.sync_copy(cnts.at[sl], zvec)
            pltpu.sync_copy(zvec, cnt_h.at[cid].at[sl])

    return k


_sc_seg_cnt = _make_sc_segment_sum(True)
_sc_seg = _make_sc_segment_sum(False)


_BLK = 1024
_GRID = NP // _BLK


def _tc_layer1(aggp, cntp, xp, w1l, b1l, w1r):
    def body(agg_ref, cnt_ref, x_ref, wl_ref, bl_ref, wr_ref, o_ref):
        agg = agg_ref[0] + agg_ref[1]
        cnt = cnt_ref[0] + cnt_ref[1]
        inv = 1.0 / jnp.maximum(cnt, 1.0)
        mean = agg * inv[:, None]
        h = lax.dot_general(mean, wl_ref[...], (((1,), (1,)), ((), ())),
                            preferred_element_type=jnp.float32)
        h = h + bl_ref[...]
        h = h + lax.dot_general(x_ref[...], wr_ref[...], (((1,), (1,)), ((), ())),
                                preferred_element_type=jnp.float32)
        o_ref[...] = jnp.maximum(h, 0.0)

    return pl.pallas_call(
        body,
        grid=(_GRID,),
        in_specs=[
            pl.BlockSpec((NC, _BLK, D), lambda i: (0, i, 0)),
            pl.BlockSpec((NC, _BLK), lambda i: (0, i)),
            pl.BlockSpec((_BLK, D), lambda i: (i, 0)),
            pl.BlockSpec((D, D), lambda i: (0, 0)),
            pl.BlockSpec((1, D), lambda i: (0, 0)),
            pl.BlockSpec((D, D), lambda i: (0, 0)),
        ],
        out_specs=pl.BlockSpec((_BLK, D), lambda i: (i, 0)),
        out_shape=jax.ShapeDtypeStruct((NP, D), jnp.float32),
    )(aggp, cntp, xp, w1l, b1l, w1r)


def _tc_layer2(aggp, cntp, h1, w2l, b2l, w2r, wo1, bo1, wo2p, bo2p):
    def body(agg_ref, cnt_ref, h1_ref, wl_ref, bl_ref, wr_ref, wo1_ref,
             bo1_ref, wo2_ref, bo2_ref, o_ref):
        agg = agg_ref[0] + agg_ref[1]
        cnt = cnt_ref[0] + cnt_ref[1]
        inv = 1.0 / jnp.maximum(cnt, 1.0)
        mean = agg * inv[:, None]
        h = lax.dot_general(mean, wl_ref[...], (((1,), (1,)), ((), ())),
                            preferred_element_type=jnp.float32)
        h = h + bl_ref[...]
        h = h + lax.dot_general(h1_ref[...], wr_ref[...], (((1,), (1,)), ((), ())),
                                preferred_element_type=jnp.float32)
        h = jnp.maximum(h, 0.0)
        h = lax.dot_general(h, wo1_ref[...], (((1,), (1,)), ((), ())),
                            preferred_element_type=jnp.float32) + bo1_ref[...]
        h = jnp.maximum(h, 0.0)
        o_ref[...] = lax.dot_general(h, wo2_ref[...], (((1,), (1,)), ((), ())),
                                     preferred_element_type=jnp.float32) + bo2_ref[...]

    return pl.pallas_call(
        body,
        grid=(_GRID,),
        in_specs=[
            pl.BlockSpec((NC, _BLK, D), lambda i: (0, i, 0)),
            pl.BlockSpec((NC, _BLK), lambda i: (0, i)),
            pl.BlockSpec((_BLK, D), lambda i: (i, 0)),
            pl.BlockSpec((D, D), lambda i: (0, 0)),
            pl.BlockSpec((1, D), lambda i: (0, 0)),
            pl.BlockSpec((D, D), lambda i: (0, 0)),
            pl.BlockSpec((D, D), lambda i: (0, 0)),
            pl.BlockSpec((1, D), lambda i: (0, 0)),
            pl.BlockSpec((D, D), lambda i: (0, 0)),
            pl.BlockSpec((1, D), lambda i: (0, 0)),
        ],
        out_specs=pl.BlockSpec((_BLK, D), lambda i: (i, 0)),
        out_shape=jax.ShapeDtypeStruct((NP, D), jnp.float32),
    )(aggp, cntp, h1, w2l, b2l, w2r, wo1, bo1, wo2p, bo2p)


def kernel(x, edge_index, W1l, b1l, W1r, W2l, b2l, W2r, Wo1, bo1, Wo2, bo2):
    src2 = edge_index[0].astype(jnp.int32).reshape(NW, EPW)
    dst3 = edge_index[1].astype(jnp.int32).reshape(NW, NCHUNK, C)
    xp = jnp.pad(x, ((0, NP - N), (0, 0)))

    wo2p = jnp.zeros((D, D), jnp.float32).at[:2].set(Wo2)
    bo2p = jnp.zeros((1, D), jnp.float32).at[0, :2].set(bo2)

    aggp1, cntp = _sc_seg_cnt(xp, src2, dst3)
    h1 = _tc_layer1(aggp1, cntp, xp, W1l, b1l.reshape(1, D), W1r)
    res2 = _sc_seg(h1, src2, dst3)
    aggp2 = res2[0] if isinstance(res2, (tuple, list)) else res2
    out = _tc_layer2(aggp2, cntp, h1, W2l, b2l.reshape(1, D), W2r,
                     Wo1, bo1.reshape(1, D), wo2p, bo2p)
    return out[:N, :2]
